# Initial kernel scaffold; baseline (speedup 1.0000x reference)
import jax
import jax.numpy as jnp
from jax import lax
from jax.experimental import pallas as pl
from jax.experimental.pallas import tpu as pltpu
from jax.experimental.pallas import tpu_sc as plsc

N = 10000
E = 320000
B = 16
D_IN = 128
D_H = 64

NC = 2
NS = 16
L = 16
NW = NC * NS
EPW = E // NW
C = 400
NCHUNK = EPW // C
NPAD = 10240
SLICE1 = NPAD // NS
ROWS_PT = N // NS
BN_SCALE = 1.0 / (1.0 + 1e-5) ** 0.5

_MESH = plsc.VectorSubcoreMesh(
    core_axis_name="c", subcore_axis_name="s", num_cores=NC, num_subcores=NS)


def _deg_body(edata_hbm, deg_out, eb, wb, zb, deg_sp):
    cid = lax.axis_index("c")
    sid = lax.axis_index("s")
    wid = cid * NS + sid

    @pl.loop(0, SLICE1 // L)
    def _(j):
        zb[pl.ds(j * L, L)] = jnp.zeros((L,), jnp.float32)

    pltpu.sync_copy(zb, deg_sp.at[pl.ds(sid * SLICE1, SLICE1)])
    plsc.subcore_barrier()

    base0 = wid * EPW

    @pl.loop(0, NCHUNK)
    def _(i):
        base = base0 + i * C
        pltpu.sync_copy(edata_hbm.at[:, pl.ds(base, C)], eb)

        @pl.loop(0, C // L)
        def _(j):
            sl = pl.ds(j * L, L)
            wb[sl] = jnp.abs(plsc.bitcast(eb[2, sl], jnp.float32))

        pltpu.sync_copy(wb, deg_sp.at[eb.at[1]], add=True)

    plsc.subcore_barrier()
    sl = pl.ds(sid * SLICE1, SLICE1)
    pltpu.sync_copy(deg_sp.at[sl], deg_out.at[cid, sl])


_deg_call = pl.kernel(
    _deg_body,
    out_type=jax.ShapeDtypeStruct((NC, NPAD), jnp.float32),
    mesh=_MESH,
    scratch_types=[
        pltpu.VMEM((3, C), jnp.int32),
        pltpu.VMEM((C,), jnp.float32),
        pltpu.VMEM((SLICE1,), jnp.float32),
        pltpu.VMEM_SHARED((NPAD,), jnp.float32),
    ],
)


def _layer_body(edata_hbm, dis_hbm, y_hbm, acc_out,
                disv, zbuf, acc_sp,
                eb0, cb0, rows0, sg0, ss0,
                eb1, cb1, rows1, sg1, ss1,
                eb2, cb2, rows2, sg2, ss2):
    cid = lax.axis_index("c")
    sid = lax.axis_index("s")
    wid = cid * NS + sid
    base0 = wid * EPW

    pltpu.sync_copy(dis_hbm, disv)

    @pl.loop(0, 125)
    def _(r):
        for j in range(D_H // L):
            zbuf[r, pl.ds(j * L, L)] = jnp.zeros((L,), jnp.float32)

    for k in range(5):
        pltpu.sync_copy(
            zbuf, acc_sp.at[pl.ds(sid * ROWS_PT + k * 125, 125), :])
    plsc.subcore_barrier()

    bufs = [(eb0, cb0, rows0, sg0, ss0),
            (eb1, cb1, rows1, sg1, ss1),
            (eb2, cb2, rows2, sg2, ss2)]

    def prefetch(i, k, wait_s):
        eb, cb, rows, sg, ss = bufs[k]
        if wait_s:
            pltpu.make_async_copy(rows, acc_sp.at[eb.at[1]], ss).wait()
        pltpu.sync_copy(edata_hbm.at[:, pl.ds(base0 + i * C, C)], eb)
        pltpu.async_copy(y_hbm.at[eb.at[0]], rows, sg)

    def process(i, k):
        eb, cb, rows, sg, ss = bufs[k]
        pltpu.make_async_copy(y_hbm.at[eb.at[0]], rows, sg).wait()

        @pl.loop(0, C // L)
        def _(j):
            sl = pl.ds(j * L, L)
            dv = plsc.load_gather(disv, [eb[0, sl]])
            cb[sl] = jnp.abs(plsc.bitcast(eb[2, sl], jnp.float32)) * dv

        @pl.loop(0, C)
        def _(e):
            cv = jnp.full((L,), cb[e], dtype=jnp.float32)
            for j in range(D_H // L):
                sl = pl.ds(j * L, L)
                rows[e, sl] = rows[e, sl] * cv

        pltpu.async_copy(rows, acc_sp.at[eb.at[1]], ss, add=True)

    prefetch(0, 0, False)
    prefetch(1, 1, False)
    process(0, 0)
    prefetch(2, 2, False)

    @pl.loop(0, 7)
    def _(g):
        i0 = 3 * g + 1
        for b in range(3):
            process(i0 + b, (1 + b) % 3)
            prefetch(i0 + b + 2, b, True)

    process(22, 1)
    prefetch(24, 0, True)
    process(23, 2)
    process(24, 0)

    for k in range(3):
        eb, cb, rows, sg, ss = bufs[k]
        pltpu.make_async_copy(rows, acc_sp.at[eb.at[1]], ss).wait()
    plsc.subcore_barrier()

    sl = pl.ds(sid * ROWS_PT, ROWS_PT)
    pltpu.sync_copy(acc_sp.at[sl, :], acc_out.at[cid, sl, :])


def _slot_scratch():
    return [
        pltpu.VMEM((3, C), jnp.int32),
        pltpu.VMEM((C,), jnp.float32),
        pltpu.VMEM((C, D_H), jnp.float32),
        pltpu.SemaphoreType.DMA,
        pltpu.SemaphoreType.DMA,
    ]


_layer_call = pl.kernel(
    _layer_body,
    out_type=jax.ShapeDtypeStruct((NC, N, D_H), jnp.float32),
    mesh=_MESH,
    scratch_types=[
        pltpu.VMEM((NPAD,), jnp.float32),
        pltpu.VMEM((125, D_H), jnp.float32),
        pltpu.VMEM_SHARED((N, D_H), jnp.float32),
    ] + _slot_scratch() + _slot_scratch() + _slot_scratch(),
)


def _tc_first_body(deg_ref, x_ref, w0_ref, dis_ref, y0_ref):
    dis_ref[...] = lax.rsqrt(1.0 + deg_ref[0:1, :] + deg_ref[1:2, :])
    y0_ref[...] = jnp.dot(x_ref[...], w0_ref[...],
                          preferred_element_type=jnp.float32)


_tc_first = pl.pallas_call(
    _tc_first_body,
    out_shape=(jax.ShapeDtypeStruct((1, NPAD), jnp.float32),
               jax.ShapeDtypeStruct((N, D_H), jnp.float32)),
)


def _tc_mid_body(acc_ref, y_ref, dis_ref, b_ref, w_ref, ynext_ref):
    dis = dis_ref[...]
    h = dis * (acc_ref[0] + acc_ref[1]) + (dis * dis) * y_ref[...] + b_ref[...]
    h = jnp.maximum(h * BN_SCALE, 0.0)
    ynext_ref[...] = jnp.dot(h, w_ref[...], preferred_element_type=jnp.float32)


_tc_mid = pl.pallas_call(
    _tc_mid_body,
    out_shape=jax.ShapeDtypeStruct((N, D_H), jnp.float32),
)


def _tc_final_body(acc_ref, y_ref, dis_ref, b_ref, batch_ref, out_ref):
    dis = dis_ref[...]
    h = dis * (acc_ref[0] + acc_ref[1]) + (dis * dis) * y_ref[...] + b_ref[...]
    h = jnp.maximum(h * BN_SCALE, 0.0)
    seg = (batch_ref[...] == lax.broadcasted_iota(jnp.int32, (B, N), 0))
    seg = seg.astype(jnp.float32)
    ssum = jnp.dot(seg, h, preferred_element_type=jnp.float32)
    cnt = jnp.sum(seg, axis=1, keepdims=True)
    out_ref[:, :D_H] = ssum / jnp.maximum(cnt, 1.0)
    out_ref[:, D_H:] = ssum


_tc_final = pl.pallas_call(
    _tc_final_body,
    out_shape=jax.ShapeDtypeStruct((B, 2 * D_H), jnp.float32),
)


def kernel(x, edge_index, edge_weight, batch, W0, b0, W1, b1, W2, b2):
    wbits = lax.bitcast_convert_type(edge_weight, jnp.int32)
    edata = jnp.concatenate(
        [edge_index, wbits[None, :]], axis=0)

    deg = _deg_call(edata)
    dis2d, y0 = _tc_first(deg, x, W0)
    dis_flat = dis2d.reshape((NPAD,))
    dis_col = dis2d[0, :N, None]

    acc0 = _layer_call(edata, dis_flat, y0)
    y1 = _tc_mid(acc0, y0, dis_col, b0.reshape(1, D_H), W1)
    acc1 = _layer_call(edata, dis_flat, y1)
    y2 = _tc_mid(acc1, y1, dis_col, b1.reshape(1, D_H), W2)
    acc2 = _layer_call(edata, dis_flat, y2)
    out = _tc_final(acc2, y2, dis_col, b2.reshape(1, D_H), batch.reshape(1, N))
    return out


# speedup vs baseline: 29.4366x; 29.4366x over previous
import jax
import jax.numpy as jnp
from jax import lax
from jax.experimental import pallas as pl
from jax.experimental.pallas import tpu as pltpu
from jax.experimental.pallas import tpu_sc as plsc

N = 10000
E = 320000
B = 16
D_IN = 128
D_H = 64

NC = 2
NS = 16
L = 16
NW = NC * NS
EPW = E // NW
C = 400
NCHUNK = EPW // C
NPAD = 10240
SLICE1 = NPAD // NS
ROWS_PT = N // NS
BN_SCALE = 1.0 / (1.0 + 1e-5) ** 0.5

_MESH = plsc.VectorSubcoreMesh(
    core_axis_name="c", subcore_axis_name="s", num_cores=NC, num_subcores=NS)
_SC_PARAMS = pltpu.CompilerParams(needs_layout_passes=False, use_tc_tiling_on_sc=False)


def _deg_body(edata_hbm, deg_out, eb, wb, zb, deg_sp):
    cid = lax.axis_index("c")
    sid = lax.axis_index("s")
    wid = cid * NS + sid

    @pl.loop(0, SLICE1 // L)
    def _(j):
        zb[pl.ds(j * L, L)] = jnp.zeros((L,), jnp.float32)

    pltpu.sync_copy(zb, deg_sp.at[pl.ds(sid * SLICE1, SLICE1)])
    plsc.subcore_barrier()

    base0 = wid * EPW

    @pl.loop(0, NCHUNK)
    def _(i):
        base = base0 + i * C
        pltpu.sync_copy(edata_hbm.at[:, pl.ds(base, C)], eb)

        @pl.loop(0, C // L)
        def _(j):
            sl = pl.ds(j * L, L)
            wb[sl] = jnp.abs(plsc.bitcast(eb[2, sl], jnp.float32))

        pltpu.sync_copy(wb, deg_sp.at[eb.at[1]], add=True)

    plsc.subcore_barrier()
    sl = pl.ds(sid * SLICE1, SLICE1)
    pltpu.sync_copy(deg_sp.at[sl], deg_out.at[cid, sl])


_deg_call = pl.kernel(
    _deg_body,
    out_type=jax.ShapeDtypeStruct((NC, NPAD), jnp.float32),
    mesh=_MESH,
    compiler_params=_SC_PARAMS,
    scratch_types=[
        pltpu.VMEM((3, C), jnp.int32),
        pltpu.VMEM((C,), jnp.float32),
        pltpu.VMEM((SLICE1,), jnp.float32),
        pltpu.VMEM_SHARED((NPAD,), jnp.float32),
    ],
)


def _layer_body(edata_hbm, dis_hbm, y_hbm, acc_out,
                disv, zbuf, acc_sp,
                eb0, rows0, sg0, ss0,
                eb1, rows1, sg1, ss1,
                eb2, rows2, sg2, ss2):
    cid = lax.axis_index("c")
    sid = lax.axis_index("s")
    wid = cid * NS + sid
    base0 = wid * EPW

    pltpu.sync_copy(dis_hbm, disv)

    @pl.loop(0, C)
    def _(r):
        for j in range(D_H // L):
            rows0[r, pl.ds(j * L, L)] = jnp.zeros((L,), jnp.float32)

    pltpu.sync_copy(rows0, acc_sp.at[pl.ds(sid * ROWS_PT, C), :])
    pltpu.sync_copy(rows0.at[pl.ds(0, ROWS_PT - C), :],
                    acc_sp.at[pl.ds(sid * ROWS_PT + C, ROWS_PT - C), :])
    plsc.subcore_barrier()

    bufs = [(eb0, rows0, sg0, ss0),
            (eb1, rows1, sg1, ss1),
            (eb2, rows2, sg2, ss2)]

    def prefetch(i, k, wait_s):
        eb, rows, sg, ss = bufs[k]
        if wait_s:
            pltpu.make_async_copy(rows, acc_sp.at[eb.at[1]], ss).wait()
        pltpu.sync_copy(edata_hbm.at[:, pl.ds(base0 + i * C, C)], eb)
        pltpu.async_copy(y_hbm.at[eb.at[0]], rows, sg)

    def process(i, k):
        eb, rows, sg, ss = bufs[k]
        pltpu.make_async_copy(y_hbm.at[eb.at[0]], rows, sg).wait()

        @plsc.parallel_loop(0, C // L, unroll=2)
        def _(g):
            sl = pl.ds(g * L, L)
            dv = plsc.load_gather(disv, [eb[0, sl]])
            cvec = jnp.abs(plsc.bitcast(eb[2, sl], jnp.float32)) * dv
            base_e = g * L
            for idx in range(L):
                cv = jnp.full((L,), cvec[idx], dtype=jnp.float32)
                e = base_e + idx
                for j in range(D_H // L):
                    sl2 = pl.ds(j * L, L)
                    rows[e, sl2] = rows[e, sl2] * cv

        pltpu.async_copy(rows, acc_sp.at[eb.at[1]], ss, add=True)

    prefetch(0, 0, False)
    prefetch(1, 1, False)
    process(0, 0)
    prefetch(2, 2, False)

    @pl.loop(0, 7)
    def _(g):
        i0 = 3 * g + 1
        for b in range(3):
            process(i0 + b, (1 + b) % 3)
            prefetch(i0 + b + 2, b, True)

    process(22, 1)
    prefetch(24, 0, True)
    process(23, 2)
    process(24, 0)

    for k in range(3):
        eb, rows, sg, ss = bufs[k]
        pltpu.make_async_copy(rows, acc_sp.at[eb.at[1]], ss).wait()
    plsc.subcore_barrier()

    sl = pl.ds(sid * ROWS_PT, ROWS_PT)
    pltpu.sync_copy(acc_sp.at[sl, :], acc_out.at[cid, sl, :])


def _slot_scratch():
    return [
        pltpu.VMEM((3, C), jnp.int32),
        pltpu.VMEM((C, D_H), jnp.float32),
        pltpu.SemaphoreType.DMA,
        pltpu.SemaphoreType.DMA,
    ]


_layer_call = pl.kernel(
    _layer_body,
    out_type=jax.ShapeDtypeStruct((NC, N, D_H), jnp.float32),
    mesh=_MESH,
    compiler_params=_SC_PARAMS,
    scratch_types=[
        pltpu.VMEM((NPAD,), jnp.float32),
        pltpu.VMEM_SHARED((N, D_H), jnp.float32),
    ] + _slot_scratch() + _slot_scratch() + _slot_scratch(),
)


def _tc_first_body(deg_ref, x_ref, w0_ref, dis_ref, y0_ref):
    dis_ref[...] = lax.rsqrt(1.0 + deg_ref[0:1, :] + deg_ref[1:2, :])
    y0_ref[...] = jnp.dot(x_ref[...], w0_ref[...],
                          preferred_element_type=jnp.float32)


_tc_first = pl.pallas_call(
    _tc_first_body,
    out_shape=(jax.ShapeDtypeStruct((1, NPAD), jnp.float32),
               jax.ShapeDtypeStruct((N, D_H), jnp.float32)),
)


def _tc_mid_body(acc_ref, y_ref, dis_ref, b_ref, w_ref, ynext_ref):
    dis = dis_ref[...]
    acc = acc_ref[0] + acc_ref[1]
    h = dis * acc + (dis * dis) * y_ref[...] + b_ref[...]
    h = jnp.maximum(h * BN_SCALE, 0.0)
    ynext_ref[...] = jnp.dot(h, w_ref[...], preferred_element_type=jnp.float32)


_tc_mid = pl.pallas_call(
    _tc_mid_body,
    out_shape=jax.ShapeDtypeStruct((N, D_H), jnp.float32),
)


def _tc_final_body(acc_ref, y_ref, dis_ref, b_ref, batch_ref, out_ref):
    dis = dis_ref[...]
    acc = acc_ref[0] + acc_ref[1]
    h = dis * acc + (dis * dis) * y_ref[...] + b_ref[...]
    h = jnp.maximum(h * BN_SCALE, 0.0)
    seg = (batch_ref[...] == lax.broadcasted_iota(jnp.int32, (B, N), 0))
    seg = seg.astype(jnp.float32)
    ssum = jnp.dot(seg, h, preferred_element_type=jnp.float32)
    cnt = jnp.sum(seg, axis=1, keepdims=True)
    out_ref[:, :D_H] = ssum / jnp.maximum(cnt, 1.0)
    out_ref[:, D_H:] = ssum


_tc_final = pl.pallas_call(
    _tc_final_body,
    out_shape=jax.ShapeDtypeStruct((B, 2 * D_H), jnp.float32),
)


def kernel(x, edge_index, edge_weight, batch, W0, b0, W1, b1, W2, b2):
    wbits = lax.bitcast_convert_type(edge_weight, jnp.int32)
    edata = jnp.concatenate(
        [edge_index, wbits[None, :]], axis=0)

    deg = _deg_call(edata)
    dis2d, y0 = _tc_first(deg, x, W0)
    dis_flat = dis2d.reshape((NPAD,))
    dis_col = dis2d[0, :N, None]

    acc0 = _layer_call(edata, dis_flat, y0)
    y1 = _tc_mid(acc0, y0, dis_col, b0.reshape(1, D_H), W1)
    acc1 = _layer_call(edata, dis_flat, y1)
    y2 = _tc_mid(acc1, y1, dis_col, b1.reshape(1, D_H), W2)
    acc2 = _layer_call(edata, dis_flat, y2)
    out = _tc_final(acc2, y2, dis_col, b2.reshape(1, D_H), batch.reshape(1, N))
    return out


# speedup vs baseline: 36.7168x; 1.2473x over previous
import jax
import jax.numpy as jnp
from jax import lax
from jax.experimental import pallas as pl
from jax.experimental.pallas import tpu as pltpu
from jax.experimental.pallas import tpu_sc as plsc

N = 10000
E = 320000
B = 16
D_IN = 128
D_H = 64

NC = 2
NS = 16
L = 16
NW = NC * NS
EPW = E // NW
C = 400
NCHUNK = EPW // C
NPAD = 10240
SLICE1 = NPAD // NS
ROWS_PT = N // NS
BN_SCALE = 1.0 / (1.0 + 1e-5) ** 0.5

_MESH = plsc.VectorSubcoreMesh(
    core_axis_name="c", subcore_axis_name="s", num_cores=NC, num_subcores=NS)
_SC_PARAMS = pltpu.CompilerParams(needs_layout_passes=False, use_tc_tiling_on_sc=False)


def _deg_body(ei_hbm, w_hbm, deg_out,
              zb, deg_sp,
              db0, wb0, si0, ss0,
              db1, wb1, si1, ss1,
              db2, wb2, si2, ss2):
    cid = lax.axis_index("c")
    sid = lax.axis_index("s")
    wid = cid * NS + sid

    @pl.loop(0, SLICE1 // L)
    def _(j):
        zb[pl.ds(j * L, L)] = jnp.zeros((L,), jnp.float32)

    pltpu.sync_copy(zb, deg_sp.at[pl.ds(sid * SLICE1, SLICE1)])
    plsc.subcore_barrier()

    base0 = wid * EPW
    bufs = [(db0, wb0, si0, ss0), (db1, wb1, si1, ss1), (db2, wb2, si2, ss2)]

    def pf(i, k, wait_s):
        db, wb, si, ss = bufs[k]
        if wait_s:
            pltpu.make_async_copy(wb, deg_sp.at[db], ss).wait()
        base = base0 + i * C
        pltpu.async_copy(ei_hbm.at[1, pl.ds(base, C)], db, si)
        pltpu.async_copy(w_hbm.at[pl.ds(base, C)], wb, si)

    def pr(i, k):
        db, wb, si, ss = bufs[k]
        base = base0 + i * C
        pltpu.make_async_copy(ei_hbm.at[1, pl.ds(base, C)], db, si).wait()
        pltpu.make_async_copy(w_hbm.at[pl.ds(base, C)], wb, si).wait()

        @pl.loop(0, C // L)
        def _(j):
            sl = pl.ds(j * L, L)
            wb[sl] = jnp.abs(wb[sl])

        pltpu.async_copy(wb, deg_sp.at[db], ss, add=True)

    pf(0, 0, False)
    pf(1, 1, False)
    pr(0, 0)
    pf(2, 2, False)

    @pl.loop(0, 7)
    def _(g):
        i0 = 3 * g + 1
        for b in range(3):
            pr(i0 + b, (1 + b) % 3)
            pf(i0 + b + 2, b, True)

    pr(22, 1)
    pf(24, 0, True)
    pr(23, 2)
    pr(24, 0)

    for k in range(3):
        db, wb, si, ss = bufs[k]
        pltpu.make_async_copy(wb, deg_sp.at[db], ss).wait()
    plsc.subcore_barrier()
    sl = pl.ds(sid * SLICE1, SLICE1)
    pltpu.sync_copy(deg_sp.at[sl], deg_out.at[cid, sl])


_deg_call = pl.kernel(
    _deg_body,
    out_type=jax.ShapeDtypeStruct((NC, NPAD), jnp.float32),
    mesh=_MESH,
    compiler_params=_SC_PARAMS,
    scratch_types=[
        pltpu.VMEM((SLICE1,), jnp.float32),
        pltpu.VMEM_SHARED((NPAD,), jnp.float32),
    ] + 3 * [
        pltpu.VMEM((C,), jnp.int32),
        pltpu.VMEM((C,), jnp.float32),
        pltpu.SemaphoreType.DMA,
        pltpu.SemaphoreType.DMA,
    ],
)


def _layer_body(ei_hbm, w_hbm, dis_hbm, y_hbm, acc_out,
                disv, acc_sp,
                eb0, wb0, rows0, si0, sg0, sh0, ss0,
                eb1, wb1, rows1, si1, sg1, sh1, ss1,
                eb2, wb2, rows2, si2, sg2, sh2, ss2):
    cid = lax.axis_index("c")
    sid = lax.axis_index("s")
    wid = cid * NS + sid
    base0 = wid * EPW

    pltpu.async_copy(dis_hbm, disv, si0)

    @pl.loop(0, C)
    def _(r):
        for j in range(D_H // L):
            rows0[r, pl.ds(j * L, L)] = jnp.zeros((L,), jnp.float32)

    pltpu.sync_copy(rows0, acc_sp.at[pl.ds(sid * ROWS_PT, C), :])
    pltpu.sync_copy(rows0.at[pl.ds(0, ROWS_PT - C), :],
                    acc_sp.at[pl.ds(sid * ROWS_PT + C, ROWS_PT - C), :])
    pltpu.make_async_copy(dis_hbm, disv, si0).wait()
    plsc.subcore_barrier()

    bufs = [(eb0, wb0, rows0, si0, sg0, sh0, ss0),
            (eb1, wb1, rows1, si1, sg1, sh1, ss1),
            (eb2, wb2, rows2, si2, sg2, sh2, ss2)]

    H = C // 2

    def pf_idx(i, k, wait_s):
        eb, wb, rows, si, sg, sh, ss = bufs[k]
        if wait_s:
            pltpu.make_async_copy(rows, acc_sp.at[eb.at[1]], ss).wait()
        base = base0 + i * C
        pltpu.async_copy(ei_hbm.at[:, pl.ds(base, C)], eb, si)
        pltpu.async_copy(w_hbm.at[pl.ds(base, C)], wb, si)

    def pf_gather(i, k):
        eb, wb, rows, si, sg, sh, ss = bufs[k]
        base = base0 + i * C
        pltpu.make_async_copy(ei_hbm.at[:, pl.ds(base, C)], eb, si).wait()
        pltpu.make_async_copy(w_hbm.at[pl.ds(base, C)], wb, si).wait()
        pltpu.async_copy(y_hbm.at[eb.at[0, pl.ds(0, H)]],
                         rows.at[pl.ds(0, H), :], sg)
        pltpu.async_copy(y_hbm.at[eb.at[0, pl.ds(H, H)]],
                         rows.at[pl.ds(H, H), :], sh)

    def process(i, k):
        eb, wb, rows, si, sg, sh, ss = bufs[k]
        pltpu.make_async_copy(y_hbm.at[eb.at[0, pl.ds(0, H)]],
                              rows.at[pl.ds(0, H), :], sg).wait()
        pltpu.make_async_copy(y_hbm.at[eb.at[0, pl.ds(H, H)]],
                              rows.at[pl.ds(H, H), :], sh).wait()

        @plsc.parallel_loop(0, C // L, unroll=4)
        def _(g):
            sl = pl.ds(g * L, L)
            dv = plsc.load_gather(disv, [eb[0, sl]])
            cvec = jnp.abs(wb[sl]) * dv
            base_e = g * L
            for idx in range(L):
                cv = jnp.full((L,), cvec[idx], dtype=jnp.float32)
                e = base_e + idx
                for j in range(D_H // L):
                    sl2 = pl.ds(j * L, L)
                    rows[e, sl2] = rows[e, sl2] * cv

        pltpu.async_copy(rows, acc_sp.at[eb.at[1]], ss, add=True)

    pf_idx(0, 0, False)
    pf_idx(1, 1, False)
    pf_gather(0, 0)
    pf_idx(2, 2, False)
    pf_gather(1, 1)
    process(0, 0)

    @pl.loop(0, 7)
    def _(g):
        i0 = 3 * g + 1
        for b in range(3):
            i = i0 + b
            pf_gather(i + 1, (2 + b) % 3)
            pf_idx(i + 2, b, True)
            process(i, (1 + b) % 3)

    pf_gather(23, 2)
    pf_idx(24, 0, True)
    process(22, 1)
    pf_gather(24, 0)
    process(23, 2)
    process(24, 0)

    for k in range(3):
        eb, wb, rows, si, sg, sh, ss = bufs[k]
        pltpu.make_async_copy(rows, acc_sp.at[eb.at[1]], ss).wait()
    plsc.subcore_barrier()

    sl = pl.ds(sid * ROWS_PT, ROWS_PT)
    pltpu.sync_copy(acc_sp.at[sl, :], acc_out.at[cid, sl, :])


def _slot_scratch():
    return [
        pltpu.VMEM((2, C), jnp.int32),
        pltpu.VMEM((C,), jnp.float32),
        pltpu.VMEM((C, D_H), jnp.float32),
        pltpu.SemaphoreType.DMA,
        pltpu.SemaphoreType.DMA,
        pltpu.SemaphoreType.DMA,
        pltpu.SemaphoreType.DMA,
    ]


_layer_call = pl.kernel(
    _layer_body,
    out_type=jax.ShapeDtypeStruct((NC, N, D_H), jnp.float32),
    mesh=_MESH,
    compiler_params=_SC_PARAMS,
    scratch_types=[
        pltpu.VMEM((NPAD,), jnp.float32),
        pltpu.VMEM_SHARED((N, D_H), jnp.float32),
    ] + _slot_scratch() + _slot_scratch() + _slot_scratch(),
)


def _tc_first_body(deg_ref, x_ref, w0_ref, dis_ref, y0_ref):
    dis_ref[...] = lax.rsqrt(1.0 + deg_ref[0:1, :] + deg_ref[1:2, :])
    y0_ref[...] = jnp.dot(x_ref[...], w0_ref[...],
                          preferred_element_type=jnp.float32)


_tc_first = pl.pallas_call(
    _tc_first_body,
    out_shape=(jax.ShapeDtypeStruct((1, NPAD), jnp.float32),
               jax.ShapeDtypeStruct((N, D_H), jnp.float32)),
)


def _tc_mid_body(acc_ref, y_ref, dis_ref, b_ref, w_ref, ynext_ref):
    dis = dis_ref[...]
    acc = acc_ref[0] + acc_ref[1]
    h = dis * acc + (dis * dis) * y_ref[...] + b_ref[...]
    h = jnp.maximum(h * BN_SCALE, 0.0)
    ynext_ref[...] = jnp.dot(h, w_ref[...], preferred_element_type=jnp.float32)


_tc_mid = pl.pallas_call(
    _tc_mid_body,
    out_shape=jax.ShapeDtypeStruct((N, D_H), jnp.float32),
)


def _tc_final_body(acc_ref, y_ref, dis_ref, b_ref, batch_ref, out_ref):
    dis = dis_ref[...]
    acc = acc_ref[0] + acc_ref[1]
    h = dis * acc + (dis * dis) * y_ref[...] + b_ref[...]
    h = jnp.maximum(h * BN_SCALE, 0.0)
    seg = (batch_ref[...] == lax.broadcasted_iota(jnp.int32, (B, N), 0))
    seg = seg.astype(jnp.float32)
    ssum = jnp.dot(seg, h, preferred_element_type=jnp.float32)
    cnt = jnp.sum(seg, axis=1, keepdims=True)
    out_ref[:, :D_H] = ssum / jnp.maximum(cnt, 1.0)
    out_ref[:, D_H:] = ssum


_tc_final = pl.pallas_call(
    _tc_final_body,
    out_shape=jax.ShapeDtypeStruct((B, 2 * D_H), jnp.float32),
)


def kernel(x, edge_index, edge_weight, batch, W0, b0, W1, b1, W2, b2):
    deg = _deg_call(edge_index, edge_weight)
    dis2d, y0 = _tc_first(deg, x, W0)
    dis_flat = dis2d.reshape((NPAD,))
    dis_col = dis2d[0, :N, None]

    acc0 = _layer_call(edge_index, edge_weight, dis_flat, y0)
    y1 = _tc_mid(acc0, y0, dis_col, b0.reshape(1, D_H), W1)
    acc1 = _layer_call(edge_index, edge_weight, dis_flat, y1)
    y2 = _tc_mid(acc1, y1, dis_col, b1.reshape(1, D_H), W2)
    acc2 = _layer_call(edge_index, edge_weight, dis_flat, y2)
    out = _tc_final(acc2, y2, dis_col, b2.reshape(1, D_H), batch.reshape(1, N))
    return out
